# baseline (speedup 1.0000x reference)
import functools

import jax
import jax.numpy as jnp
from jax import lax
from jax.experimental import pallas as pl
from jax.experimental.pallas import tpu as pltpu
from jax.experimental.pallas import tpu_sc as plsc

B = 64
W = 512
BAND = 32 * W
HALF = BAND // 2

NUM_CORES = 2
NUM_SUBCORES = 16
NW = NUM_CORES * NUM_SUBCORES
TASKS = B * 16
TPW = TASKS // NW

N_IN = 3
N_OUT = 2

_mesh = plsc.VectorSubcoreMesh(
    core_axis_name="c", subcore_axis_name="s",
    num_cores=NUM_CORES, num_subcores=NUM_SUBCORES)


@functools.partial(
    pl.kernel,
    out_type=jax.ShapeDtypeStruct((B, 16, BAND), jnp.float32),
    mesh=_mesh,
    compiler_params=pltpu.CompilerParams(
        use_tc_tiling_on_sc=False, needs_layout_passes=False),
    scratch_types=(
        [pltpu.VMEM((BAND,), jnp.float32)] * (N_IN + N_OUT)
        + [pltpu.SemaphoreType.DMA] * (N_IN + N_OUT)
    ),
)
def _unweave(in_hbm, out_hbm, *refs):
    ins = list(refs[:N_IN])
    outs = list(refs[N_IN:N_IN + N_OUT])
    isems = list(refs[N_IN + N_OUT:2 * N_IN + N_OUT])
    osems = list(refs[2 * N_IN + N_OUT:])

    cid = lax.axis_index("c")
    sid = lax.axis_index("s")
    wid = sid * NUM_CORES + cid

    lane = lax.iota(jnp.int32, 16)
    c_lane = lane % 4
    flatpat = (c_lane // 2) * (16 * W) + (c_lane % 2) * 16 + lane // 4
    pats = [flatpat + (32 * (r >> 2) + 4 * (r & 3)) for r in range(8)]

    def hbm_in(t):
        task = t * NW + wid
        return in_hbm.at[task // 16, task % 16]

    def hbm_out(t):
        task = t * NW + wid
        return out_hbm.at[task // 16, task % 16]

    in_desc = [None] * N_IN
    out_desc = [None] * N_OUT
    for u in range(min(N_IN - 1, TPW)):
        in_desc[u] = pltpu.async_copy(hbm_in(u), ins[u], isems[u])
    for t in range(TPW):
        isl = t % N_IN
        osl = t % N_OUT
        u = t + N_IN - 1
        if u < TPW:
            in_desc[u % N_IN] = pltpu.async_copy(
                hbm_in(u), ins[u % N_IN], isems[u % N_IN])
        in_desc[isl].wait()
        if out_desc[osl] is not None:
            out_desc[osl].wait()
        ibuf = ins[isl]
        obuf = outs[osl]

        @plsc.parallel_loop(0, 1024, step=1, unroll=8)
        def _chunk(m):
            off = (m >> 6) * W + (m & 3) * 4 + ((m >> 2) & 15) * 32
            vals = plsc.load_gather(ibuf, [flatpat + off])
            obuf[pl.ds(m * 16, 16)] = vals

        out_desc[osl] = pltpu.async_copy(obuf, hbm_out(t), osems[osl])
    for d in out_desc:
        if d is not None:
            d.wait()


def kernel(image):
    img = jnp.reshape(image, (B, 16, BAND))
    out = _unweave(img)
    return jnp.reshape(out, (B, 256, 256, 4))
